# baseline (speedup 1.0000x reference)
import jax
import jax.numpy as jnp
from jax.experimental import pallas as pl
from jax.experimental.pallas import tpu as pltpu

EPS = 0.1
N_ITERS = 50
K = 8.0

B, N, D = 4, 8192, 768
KSEL = 512
N_CHUNK = 1024


def _matvec_body(x_ref, rt_ref, s_ref):
    x = x_ref[:, :, :].reshape(B * N_CHUNK, D)
    rt = rt_ref[0]
    s = jax.lax.dot_general(
        x, rt[:, None],
        dimension_numbers=(((1,), (0,)), ((), ())),
        preferred_element_type=jnp.float32,
        precision=jax.lax.Precision.HIGHEST,
    )
    s_ref[:, :] = s.reshape(B, N_CHUNK)


def _iters_body(s_ref, a_ref):
    s = s_ref[:, :]
    constant = EPS * jnp.log(K)

    def lse_a(b):
        z = (s + b) / EPS
        m = jnp.max(z, axis=-1, keepdims=True)
        lse = jnp.log(jnp.sum(jnp.exp(z - m), axis=-1, keepdims=True)) + m
        return constant - EPS * lse

    def one_iter(_, b):
        return -jax.nn.relu(s + lse_a(b))

    b = jax.lax.fori_loop(0, N_ITERS - 1, one_iter, -jax.nn.relu(s))
    a_ref[:, :] = lse_a(b)


def _cumsum_lanes(x):
    n = x.shape[1]
    sh = 1
    while sh < n:
        shifted = jnp.concatenate(
            [jnp.zeros((x.shape[0], sh), x.dtype), x[:, :n - sh]], axis=1)
        x = x + shifted
        sh *= 2
    return x


def _topk_body(sc_ref, idx_ref, work_ref, ceq_ref):
    sc = sc_ref[:, :]
    sb = jax.lax.bitcast_convert_type(sc, jnp.int32)
    iota_n = jax.lax.broadcasted_iota(jnp.int32, (B, N), 1)

    def bisect_step(_, carry):
        lo, hi = carry
        mid = (lo + hi) // 2
        cnt = jnp.sum(jnp.where(sb >= mid, 1, 0), axis=1, keepdims=True)
        ok = cnt >= KSEL
        return (jnp.where(ok, mid, lo), jnp.where(ok, hi, mid))

    lo0 = jnp.zeros((B, 1), jnp.int32)
    hi0 = jnp.full((B, 1), 0x3F800001, jnp.int32)
    lo, _ = jax.lax.fori_loop(0, 31, bisect_step, (lo0, hi0))
    t = lo

    mask_gt = sb > t
    g = jnp.sum(jnp.where(mask_gt, 1, 0), axis=1, keepdims=True)

    work_ref[:, :] = jnp.where(mask_gt, sc, -1.0)
    max_g = jnp.max(g)
    slot_iota = jax.lax.broadcasted_iota(jnp.int32, (B, KSEL), 1)

    def extract_step(p, gslot):
        w = work_ref[:, :]
        m = jnp.max(w, axis=1, keepdims=True)
        amin = jnp.min(jnp.where(w == m, iota_n, N), axis=1, keepdims=True)
        work_ref[:, :] = jnp.where(iota_n == amin, -1.0, w)
        return jnp.where(slot_iota == p, amin, gslot)

    gslot = jax.lax.fori_loop(
        0, max_g, extract_step, jnp.zeros((B, KSEL), jnp.int32))

    ceq_ref[:, :] = _cumsum_lanes(jnp.where(sb == t, 1, 0))
    lim = slot_iota - g
    CHN = 512

    def nchunk_step(j, acc):
        cc = ceq_ref[:, pl.ds(j * CHN, CHN)]
        part = jnp.sum(
            jnp.where(cc[:, None, :] <= lim[:, :, None], 1, 0), axis=2)
        return acc + part

    eqpos = jax.lax.fori_loop(
        0, N // CHN, nchunk_step, jnp.zeros((B, KSEL), jnp.int32))

    idx_ref[:, :] = jnp.where(slot_iota < g, gslot, eqpos)


def _compute(x, rt):
    s = pl.pallas_call(
        _matvec_body,
        grid=(N // N_CHUNK,),
        in_specs=[
            pl.BlockSpec((B, N_CHUNK, D), lambda j: (0, j, 0)),
            pl.BlockSpec((1, D), lambda j: (0, 0)),
        ],
        out_specs=pl.BlockSpec((B, N_CHUNK), lambda j: (0, j)),
        out_shape=jax.ShapeDtypeStruct((B, N), jnp.float32),
    )(x, rt[None, :])

    a = pl.pallas_call(
        _iters_body,
        out_shape=jax.ShapeDtypeStruct((B, 1), jnp.float32),
    )(s)

    bfin = -jax.nn.relu(s + a)
    scores = jnp.exp((s + a + bfin) / EPS)

    idx = pl.pallas_call(
        _topk_body,
        out_shape=jax.ShapeDtypeStruct((B, KSEL), jnp.int32),
        scratch_shapes=[pltpu.VMEM((B, N), jnp.float32),
                        pltpu.VMEM((B, N), jnp.int32)],
    )(scores)
    return idx


def kernel(x, routing_token, num_tokens):
    n = x.shape[-2]
    num_tokens = jnp.minimum(num_tokens, n)
    idx = _compute(x, routing_token)
    valid = jnp.arange(KSEL) < num_tokens
    sel_scores = jnp.broadcast_to(
        jnp.where(valid, 1.0, 0.0).astype(jnp.float32), (x.shape[0], KSEL))
    sel_idx = jnp.where(valid, idx, 0)
    return (sel_scores, sel_idx)


# speedup vs baseline: 1.8393x; 1.8393x over previous
import jax
import jax.numpy as jnp
from jax.experimental import pallas as pl
from jax.experimental.pallas import tpu as pltpu

EPS = 0.1
N_ITERS = 50
K = 8.0

B, N, D = 4, 8192, 768
KSEL = 512
N_CHUNK = 1024


def _matvec_body(x_ref, rt_ref, s_ref):
    x = x_ref[:, :, :].reshape(B * N_CHUNK, D)
    rt = rt_ref[0]
    s = jax.lax.dot_general(
        x, rt[:, None],
        dimension_numbers=(((1,), (0,)), ((), ())),
        preferred_element_type=jnp.float32,
        precision=jax.lax.Precision.DEFAULT,
    )
    s_ref[:, :] = s.reshape(B, N_CHUNK)


def _iters_body(s_ref, a_ref):
    s = s_ref[:, :]
    constant = EPS * jnp.log(K)
    inv_eps = jnp.float32(1.0 / EPS)

    def one_iter(_, a):
        u = jnp.minimum(s + a, 0.0) * inv_eps
        ssum = jnp.sum(jnp.exp(u), axis=-1, keepdims=True)
        return a + (constant - EPS * jnp.log(ssum))

    a = jax.lax.fori_loop(
        0, N_ITERS, one_iter, jnp.zeros((B, 1), jnp.float32))
    a_ref[:, :] = a


def _cumsum_lanes(x):
    n = x.shape[1]
    sh = 1
    while sh < n:
        shifted = jnp.concatenate(
            [jnp.zeros((x.shape[0], sh), x.dtype), x[:, :n - sh]], axis=1)
        x = x + shifted
        sh *= 2
    return x


def _topk_body(sc_ref, idx_ref, work_ref, ceq_ref):
    sc = sc_ref[:, :]
    sb = jax.lax.bitcast_convert_type(sc, jnp.int32)
    iota_n = jax.lax.broadcasted_iota(jnp.int32, (B, N), 1)

    def bisect_step(_, carry):
        lo, hi = carry
        mid = (lo + hi) // 2
        cnt = jnp.sum(jnp.where(sb >= mid, 1, 0), axis=1, keepdims=True)
        ok = cnt >= KSEL
        return (jnp.where(ok, mid, lo), jnp.where(ok, hi, mid))

    lo0 = jnp.zeros((B, 1), jnp.int32)
    hi0 = jnp.full((B, 1), 0x3F800001, jnp.int32)
    lo, _ = jax.lax.fori_loop(0, 31, bisect_step, (lo0, hi0))
    t = lo

    mask_gt = sb > t
    g = jnp.sum(jnp.where(mask_gt, 1, 0), axis=1, keepdims=True)

    work_ref[:, :] = jnp.where(mask_gt, sc, -1.0)
    max_g = jnp.max(g)
    slot_iota = jax.lax.broadcasted_iota(jnp.int32, (B, KSEL), 1)

    def extract_step(p, gslot):
        w = work_ref[:, :]
        m = jnp.max(w, axis=1, keepdims=True)
        amin = jnp.min(jnp.where(w == m, iota_n, N), axis=1, keepdims=True)
        work_ref[:, :] = jnp.where(iota_n == amin, -1.0, w)
        return jnp.where(slot_iota == p, amin, gslot)

    gslot = jax.lax.fori_loop(
        0, max_g, extract_step, jnp.zeros((B, KSEL), jnp.int32))

    ceq_ref[:, :] = _cumsum_lanes(jnp.where(sb == t, 1, 0))
    lim = slot_iota - g
    CHN = 512

    def nchunk_step(j, acc):
        cc = ceq_ref[:, pl.ds(j * CHN, CHN)]
        part = jnp.sum(
            jnp.where(cc[:, None, :] <= lim[:, :, None], 1, 0), axis=2)
        return acc + part

    eqpos = jax.lax.fori_loop(
        0, N // CHN, nchunk_step, jnp.zeros((B, KSEL), jnp.int32))

    idx_ref[:, :] = jnp.where(slot_iota < g, gslot, eqpos)


def _compute(x, rt):
    s = pl.pallas_call(
        _matvec_body,
        grid=(N // N_CHUNK,),
        in_specs=[
            pl.BlockSpec((B, N_CHUNK, D), lambda j: (0, j, 0)),
            pl.BlockSpec((1, D), lambda j: (0, 0)),
        ],
        out_specs=pl.BlockSpec((B, N_CHUNK), lambda j: (0, j)),
        out_shape=jax.ShapeDtypeStruct((B, N), jnp.float32),
    )(x, rt[None, :])

    a = pl.pallas_call(
        _iters_body,
        out_shape=jax.ShapeDtypeStruct((B, 1), jnp.float32),
    )(s)

    bfin = -jax.nn.relu(s + a)
    scores = jnp.exp((s + a + bfin) / EPS)

    idx = pl.pallas_call(
        _topk_body,
        out_shape=jax.ShapeDtypeStruct((B, KSEL), jnp.int32),
        scratch_shapes=[pltpu.VMEM((B, N), jnp.float32),
                        pltpu.VMEM((B, N), jnp.int32)],
    )(scores)
    return idx


def kernel(x, routing_token, num_tokens):
    n = x.shape[-2]
    num_tokens = jnp.minimum(num_tokens, n)
    idx = _compute(x, routing_token)
    valid = jnp.arange(KSEL) < num_tokens
    sel_scores = jnp.broadcast_to(
        jnp.where(valid, 1.0, 0.0).astype(jnp.float32), (x.shape[0], KSEL))
    sel_idx = jnp.where(valid, idx, 0)
    return (sel_scores, sel_idx)


# speedup vs baseline: 2.0640x; 1.1221x over previous
import jax
import jax.numpy as jnp
from jax.experimental import pallas as pl
from jax.experimental.pallas import tpu as pltpu

EPS = 0.1
N_ITERS = 50
K = 8.0

B, N, D = 4, 8192, 768
KSEL = 512
N_CHUNK = 1024


def _matvec_body(x_ref, rt_ref, s_ref):
    x = x_ref[:, :, :].reshape(B * N_CHUNK, D)
    rt = rt_ref[0]
    s = jax.lax.dot_general(
        x, rt[:, None],
        dimension_numbers=(((1,), (0,)), ((), ())),
        preferred_element_type=jnp.float32,
        precision=jax.lax.Precision.DEFAULT,
    )
    s_ref[:, :] = s.reshape(B, N_CHUNK)


def _iters_body(s_ref, a_ref):
    s = s_ref[:, :]
    constant = EPS * jnp.log(K)
    inv_eps = jnp.float32(1.0 / EPS)

    def one_iter(_, a):
        u = jnp.minimum(s + a, 0.0) * inv_eps
        ssum = jnp.sum(jnp.exp(u), axis=-1, keepdims=True)
        return a + (constant - EPS * jnp.log(ssum))

    a = jax.lax.fori_loop(
        0, N_ITERS, one_iter, jnp.zeros((B, 1), jnp.float32))
    a_ref[:, :] = a


def _cumsum_lanes(x):
    n = x.shape[1]
    sh = 1
    while sh < n:
        shifted = jnp.concatenate(
            [jnp.zeros((x.shape[0], sh), x.dtype), x[:, :n - sh]], axis=1)
        x = x + shifted
        sh *= 2
    return x


def _topk_body(sc_ref, idx_ref, work_ref, ceq_ref):
    sc = sc_ref[:, :]
    sb = jax.lax.bitcast_convert_type(sc, jnp.int32)
    iota_n = jax.lax.broadcasted_iota(jnp.int32, (B, N), 1)

    cnt_pos = jnp.sum(jnp.where(sb > 0, 1, 0), axis=1, keepdims=True)

    def bisect_all(_):
        def bisect_step(_, carry):
            lo, hi = carry
            mid = (lo + hi) // 2
            cnt = jnp.sum(jnp.where(sb >= mid, 1, 0), axis=1, keepdims=True)
            ok = cnt >= KSEL
            return (jnp.where(ok, mid, lo), jnp.where(ok, hi, mid))

        lo0 = jnp.zeros((B, 1), jnp.int32)
        hi0 = jnp.full((B, 1), 0x3F800001, jnp.int32)
        lo, _ = jax.lax.fori_loop(0, 31, bisect_step, (lo0, hi0))
        return lo

    t = jax.lax.cond(
        jnp.all(cnt_pos < KSEL),
        lambda _: jnp.zeros((B, 1), jnp.int32),
        bisect_all,
        operand=0,
    )

    mask_gt = sb > t
    g = jnp.sum(jnp.where(mask_gt, 1, 0), axis=1, keepdims=True)

    work_ref[:, :] = jnp.where(mask_gt, sc, -1.0)
    max_g = jnp.max(g)
    slot_iota = jax.lax.broadcasted_iota(jnp.int32, (B, KSEL), 1)

    def extract_step(p, gslot):
        w = work_ref[:, :]
        m = jnp.max(w, axis=1, keepdims=True)
        amin = jnp.min(jnp.where(w == m, iota_n, N), axis=1, keepdims=True)
        work_ref[:, :] = jnp.where(iota_n == amin, -1.0, w)
        return jnp.where(slot_iota == p, amin, gslot)

    gslot = jax.lax.fori_loop(
        0, max_g, extract_step, jnp.zeros((B, KSEL), jnp.int32))

    ceq_ref[:, :] = _cumsum_lanes(jnp.where(sb == t, 1, 0))
    lim = slot_iota - g
    CHN = 512

    def nchunk_step(j, acc):
        cc = ceq_ref[:, pl.ds(j * CHN, CHN)]
        part = jnp.sum(
            jnp.where(cc[:, None, :] <= lim[:, :, None], 1, 0), axis=2)
        return acc + part

    pmax = jnp.max(jnp.sum(jnp.where(ceq_ref[:, :] <= KSEL, 1, 0), axis=1))
    nchunks = (pmax + CHN - 1) // CHN
    eqpos = jax.lax.fori_loop(
        0, nchunks, nchunk_step, jnp.zeros((B, KSEL), jnp.int32))

    idx_ref[:, :] = jnp.where(slot_iota < g, gslot, eqpos)


def _compute(x, rt):
    s = pl.pallas_call(
        _matvec_body,
        grid=(N // N_CHUNK,),
        in_specs=[
            pl.BlockSpec((B, N_CHUNK, D), lambda j: (0, j, 0)),
            pl.BlockSpec((1, D), lambda j: (0, 0)),
        ],
        out_specs=pl.BlockSpec((B, N_CHUNK), lambda j: (0, j)),
        out_shape=jax.ShapeDtypeStruct((B, N), jnp.float32),
    )(x, rt[None, :])

    a = pl.pallas_call(
        _iters_body,
        out_shape=jax.ShapeDtypeStruct((B, 1), jnp.float32),
    )(s)

    bfin = -jax.nn.relu(s + a)
    scores = jnp.exp((s + a + bfin) / EPS)

    idx = pl.pallas_call(
        _topk_body,
        out_shape=jax.ShapeDtypeStruct((B, KSEL), jnp.int32),
        scratch_shapes=[pltpu.VMEM((B, N), jnp.float32),
                        pltpu.VMEM((B, N), jnp.int32)],
    )(scores)
    return idx


def kernel(x, routing_token, num_tokens):
    n = x.shape[-2]
    num_tokens = jnp.minimum(num_tokens, n)
    idx = _compute(x, routing_token)
    valid = jnp.arange(KSEL) < num_tokens
    sel_scores = jnp.broadcast_to(
        jnp.where(valid, 1.0, 0.0).astype(jnp.float32), (x.shape[0], KSEL))
    sel_idx = jnp.where(valid, idx, 0)
    return (sel_scores, sel_idx)
